# baseline (speedup 1.0000x reference)
import jax
import jax.numpy as jnp
from jax import lax
from jax.experimental import pallas as pl
from jax.experimental.pallas import tpu as pltpu
from jax.experimental.pallas import tpu_sc as plsc

N = 10000
E = 320000
D = 128
HALF = 64
NT = 16
EPT = E // NT
K = 80
BE = 4000
BCH = BE // K
NB = EPT // BE
RPT = N // NT


def _zero_rows(src_ref, acc, base, nrows):
    done = 0
    while done < nrows:
        n = min(K, nrows - done)
        pltpu.sync_copy(src_ref.at[pl.ds(0, n)],
                        acc.at[pl.ds(base + done, n)])
        done += n


def _sc_body(nf0, nf1, ed, mean_o, av_o, dx_o, deg_o,
             edb, g0, g1, w0, w1, deg_loc,
             sg0, sg1, ss0, ss1, acc):
    c = lax.axis_index("c")
    s = lax.axis_index("s")

    zeros16 = jnp.zeros((16,), jnp.float32)
    ones16 = jnp.ones((16,), jnp.float32)
    row0 = s * RPT
    col0 = c * HALF
    ebase = s * EPT
    gs = (g0, g1)
    ws = (w0, w1)
    sgs = (sg0, sg1)
    sss = (ss0, ss1)

    def start_gather(idx_ref, gp, semp):
        @pl.when(c == 0)
        def _():
            pltpu.async_copy(nf0.at[idx_ref], gp, semp)

        @pl.when(c == 1)
        def _():
            pltpu.async_copy(nf1.at[idx_ref], gp, semp)

    def drain(dst_ref, sem):
        pltpu.make_async_copy(nf0.at[pl.ds(0, K)], dst_ref, sem).wait()

    def drain_scatters(r, p):
        drain(gs[p], sss[p])
        if r == 0:
            drain(ws[p], sss[p])

    def z_g(i, _):
        for j in range(HALF // 16):
            g0[i, pl.ds(j * 16, 16)] = zeros16
        return 0
    lax.fori_loop(0, K, z_g, 0)

    def z_deg(i, _):
        deg_loc[pl.ds(i * 16, 16)] = zeros16
        return 0
    lax.fori_loop(0, N // 16, z_deg, 0)

    for r in range(2):
        _zero_rows(g0, acc, row0, RPT)
        if r == 0:
            _zero_rows(g0, acc, N + row0, RPT)
        plsc.subcore_barrier()

        def half_step(k, p):
            gp = gs[p]
            wp = ws[p]

            drain(gp, sgs[p])

            base = k * K

            def edge16(t, _):
                o16 = base + t * 16
                fvec = plsc.bitcast(edb[3, pl.ds(o16, 16)], jnp.float32)
                if r == 0:
                    fvec = jnp.abs(fvec)
                    dvec = edb[1, pl.ds(o16, 16)]
                    plsc.addupdate_scatter(deg_loc, [dvec], ones16)
                for u in range(16):
                    fv = fvec[u]
                    i = t * 16 + u
                    for j in range(HALF // 16):
                        x = gp[i, pl.ds(j * 16, 16)]
                        wp[i, pl.ds(j * 16, 16)] = x * fv
                return 0
            lax.fori_loop(0, K // 16, edge16, 0)

            @pl.when(k >= 1)
            def _():
                drain_scatters(r, 1 - p)

            @pl.when(k <= BCH - 2)
            def _():
                start_gather(edb.at[0, pl.ds((k + 1) * K, K)],
                             gs[1 - p], sgs[1 - p])

            if r == 0:
                pltpu.async_copy(gp, acc.at[edb.at[1, pl.ds(base, K)]],
                                 sss[p], add=True)
                pltpu.async_copy(wp, acc.at[edb.at[2, pl.ds(base, K)]],
                                 sss[p], add=True)
            else:
                pltpu.async_copy(wp, acc.at[edb.at[1, pl.ds(base, K)]],
                                 sss[p], add=True)

        def block(nb, _):
            @pl.when(nb >= 1)
            def _():
                drain_scatters(r, 1)

            pltpu.sync_copy(ed.at[:, pl.ds(ebase + nb * BE, BE)], edb)
            start_gather(edb.at[0, pl.ds(0, K)], g0, sg0)

            def dstep(ks, _):
                half_step(ks * 2, 0)
                half_step(ks * 2 + 1, 1)
                return 0
            lax.fori_loop(0, BCH // 2, dstep, 0)
            return 0
        lax.fori_loop(0, NB, block, 0)

        drain_scatters(r, 1)

        plsc.subcore_barrier()

        if r == 0:
            pltpu.sync_copy(acc.at[pl.ds(row0, RPT)],
                            mean_o.at[pl.ds(row0, RPT), pl.ds(col0, HALF)])
            pltpu.sync_copy(acc.at[pl.ds(N + row0, RPT)],
                            av_o.at[pl.ds(row0, RPT), pl.ds(col0, HALF)])
            pltpu.sync_copy(deg_loc, deg_o.at[c * NT + s])
            def z_g2(i, _):
                for j in range(HALF // 16):
                    g0[i, pl.ds(j * 16, 16)] = zeros16
                return 0
            lax.fori_loop(0, K, z_g2, 0)
        else:
            pltpu.sync_copy(acc.at[pl.ds(row0, RPT)],
                            dx_o.at[pl.ds(row0, RPT), pl.ds(col0, HALF)])


def _sc_aggregate(nf0, nf1, ed):
    mesh = plsc.VectorSubcoreMesh(core_axis_name="c", subcore_axis_name="s")
    f32 = jnp.float32
    kern = pl.kernel(
        _sc_body,
        mesh=mesh,
        compiler_params=pltpu.CompilerParams(use_tc_tiling_on_sc=False,
                                             needs_layout_passes=False),
        out_type=(
            jax.ShapeDtypeStruct((N, D), f32),
            jax.ShapeDtypeStruct((N, D), f32),
            jax.ShapeDtypeStruct((N, D), f32),
            jax.ShapeDtypeStruct((2 * NT, N), f32),
        ),
        scratch_types=[
            pltpu.VMEM((4, BE), jnp.int32),
            pltpu.VMEM((K, HALF), f32),
            pltpu.VMEM((K, HALF), f32),
            pltpu.VMEM((K, HALF), f32),
            pltpu.VMEM((K, HALF), f32),
            pltpu.VMEM((N,), f32),
            pltpu.SemaphoreType.DMA,
            pltpu.SemaphoreType.DMA,
            pltpu.SemaphoreType.DMA,
            pltpu.SemaphoreType.DMA,
            pltpu.VMEM_SHARED((2 * N, HALF), f32),
        ],
    )
    return kern(nf0, nf1, ed)


def _tc_body(nf, ms, avs, dxs, degp, fdig, normn, w, b, gamma, beta, out):
    deg = jnp.sum(degp[...], axis=1, keepdims=True) * 0.5
    deg = jnp.maximum(deg, 1.0)
    agg_mean = ms[...] / deg
    agg_dx = dxs[...] - fdig[...] * nf[...]
    h = (jnp.dot(nf[...], w[0:D], preferred_element_type=jnp.float32)
         + jnp.dot(agg_mean, w[D:2 * D], preferred_element_type=jnp.float32)
         + jnp.dot(avs[...], w[2 * D:3 * D], preferred_element_type=jnp.float32)
         + jnp.dot(agg_dx, w[3 * D:4 * D], preferred_element_type=jnp.float32)
         + b[...])
    h = h * normn[...]
    mu = jnp.mean(h, axis=0, keepdims=True)
    var = jnp.mean((h - mu) * (h - mu), axis=0, keepdims=True)
    h = (h - mu) * lax.rsqrt(var + 1e-5) * gamma[...] + beta[...]
    out[...] = jnp.maximum(h, 0.0)


@jax.jit
def kernel(node_fts, edge_fts, edge_index, F_norm_edge, F_dig, node_deg_vec,
           norm_n, W, b, gamma, beta):
    del edge_fts, node_deg_vec
    src = edge_index[0]
    dst = edge_index[1]
    fbits = lax.bitcast_convert_type(F_norm_edge[:, 0], jnp.int32)
    ed = jnp.stack([src, dst, dst + N, fbits])
    nf0 = node_fts[:, :HALF]
    nf1 = node_fts[:, HALF:]

    ms, avs, dxs, degp = _sc_aggregate(nf0, nf1, ed)

    out = pl.pallas_call(
        _tc_body,
        out_shape=jax.ShapeDtypeStruct((N, D), jnp.float32),
    )(node_fts, ms, avs, dxs, degp.T, F_dig, norm_n,
      W, b.reshape(1, D), gamma.reshape(1, D), beta.reshape(1, D))
    return out


# speedup vs baseline: 1.3713x; 1.3713x over previous
import jax
import jax.numpy as jnp
from jax import lax
from jax.experimental import pallas as pl
from jax.experimental.pallas import tpu as pltpu
from jax.experimental.pallas import tpu_sc as plsc

N = 10000
E = 320000
D = 128
HALF = 64
NT = 16
EPT = E // NT
K = 80
BE = 4000
BCH = BE // K
NB = EPT // BE
RPT = N // NT


def _zero_rows(src_ref, acc, base, nrows):
    done = 0
    while done < nrows:
        n = min(K, nrows - done)
        pltpu.sync_copy(src_ref.at[pl.ds(0, n)],
                        acc.at[pl.ds(base + done, n)])
        done += n


def _sc_body(nf0, nf1, ed, mean_o, av_o, dx_o, deg_o,
             edb, g0, g1, w0, w1, deg_loc,
             sg0, sg1, ss0, ss1, acc):
    c = lax.axis_index("c")
    s = lax.axis_index("s")

    zeros16 = jnp.zeros((16,), jnp.float32)
    ones16 = jnp.ones((16,), jnp.float32)
    row0 = s * RPT
    col0 = c * HALF
    ebase = s * EPT
    gs = (g0, g1)
    ws = (w0, w1)
    sgs = (sg0, sg1)
    sss = (ss0, ss1)

    def start_gather(idx_ref, gp, semp):
        @pl.when(c == 0)
        def _():
            pltpu.async_copy(nf0.at[idx_ref], gp, semp)

        @pl.when(c == 1)
        def _():
            pltpu.async_copy(nf1.at[idx_ref], gp, semp)

    def drain(dst_ref, sem):
        pltpu.make_async_copy(nf0.at[pl.ds(0, K)], dst_ref, sem).wait()

    def drain_scatters(r, p):
        drain(gs[p], sss[p])
        if r == 0:
            drain(ws[p], sss[p])

    def z_g(i, _):
        for j in range(HALF // 16):
            g0[i, pl.ds(j * 16, 16)] = zeros16
        return 0
    lax.fori_loop(0, K, z_g, 0)

    def z_deg(i, _):
        deg_loc[pl.ds(i * 16, 16)] = zeros16
        return 0
    lax.fori_loop(0, N // 16, z_deg, 0)

    for r in range(2):
        _zero_rows(g0, acc, row0, RPT)
        if r == 0:
            _zero_rows(g0, acc, N + row0, RPT)
        plsc.subcore_barrier()

        def half_step(k, p):
            gp = gs[p]
            wp = ws[p]

            @pl.when(k >= 1)
            def _():
                drain_scatters(r, 1 - p)

            @pl.when(k <= BCH - 2)
            def _():
                start_gather(edb.at[0, pl.ds((k + 1) * K, K)],
                             gs[1 - p], sgs[1 - p])

            drain(gp, sgs[p])

            base = k * K

            if r == 0:
                def degv(t, _):
                    dvec = edb[1, pl.ds(base + t * 16, 16)]
                    plsc.addupdate_scatter(deg_loc, [dvec], ones16)
                    return 0
                lax.fori_loop(0, K // 16, degv, 0)

            @plsc.parallel_loop(0, K // 16, unroll=2)
            def edge16(t):
                fvec = plsc.bitcast(edb[3, pl.ds(base + t * 16, 16)],
                                    jnp.float32)
                if r == 0:
                    fvec = jnp.abs(fvec)
                for u in range(16):
                    fv = fvec[u]
                    i = t * 16 + u
                    for j in range(HALF // 16):
                        x = gp[i, pl.ds(j * 16, 16)]
                        wp[i, pl.ds(j * 16, 16)] = x * fv

            if r == 0:
                pltpu.async_copy(gp, acc.at[edb.at[1, pl.ds(base, K)]],
                                 sss[p], add=True)
                pltpu.async_copy(wp, acc.at[edb.at[2, pl.ds(base, K)]],
                                 sss[p], add=True)
            else:
                pltpu.async_copy(wp, acc.at[edb.at[1, pl.ds(base, K)]],
                                 sss[p], add=True)

        def block(nb, _):
            @pl.when(nb >= 1)
            def _():
                drain_scatters(r, 1)

            pltpu.sync_copy(ed.at[:, pl.ds(ebase + nb * BE, BE)], edb)
            start_gather(edb.at[0, pl.ds(0, K)], g0, sg0)

            def dstep(ks, _):
                half_step(ks * 2, 0)
                half_step(ks * 2 + 1, 1)
                return 0
            lax.fori_loop(0, BCH // 2, dstep, 0)
            return 0
        lax.fori_loop(0, NB, block, 0)

        drain_scatters(r, 1)

        plsc.subcore_barrier()

        if r == 0:
            pltpu.sync_copy(acc.at[pl.ds(row0, RPT)],
                            mean_o.at[pl.ds(row0, RPT), pl.ds(col0, HALF)])
            pltpu.sync_copy(acc.at[pl.ds(N + row0, RPT)],
                            av_o.at[pl.ds(row0, RPT), pl.ds(col0, HALF)])
            pltpu.sync_copy(deg_loc, deg_o.at[c * NT + s])
            def z_g2(i, _):
                for j in range(HALF // 16):
                    g0[i, pl.ds(j * 16, 16)] = zeros16
                return 0
            lax.fori_loop(0, K, z_g2, 0)
        else:
            pltpu.sync_copy(acc.at[pl.ds(row0, RPT)],
                            dx_o.at[pl.ds(row0, RPT), pl.ds(col0, HALF)])


def _sc_aggregate(nf0, nf1, ed):
    mesh = plsc.VectorSubcoreMesh(core_axis_name="c", subcore_axis_name="s")
    f32 = jnp.float32
    kern = pl.kernel(
        _sc_body,
        mesh=mesh,
        compiler_params=pltpu.CompilerParams(use_tc_tiling_on_sc=False,
                                             needs_layout_passes=False),
        out_type=(
            jax.ShapeDtypeStruct((N, D), f32),
            jax.ShapeDtypeStruct((N, D), f32),
            jax.ShapeDtypeStruct((N, D), f32),
            jax.ShapeDtypeStruct((2 * NT, N), f32),
        ),
        scratch_types=[
            pltpu.VMEM((4, BE), jnp.int32),
            pltpu.VMEM((K, HALF), f32),
            pltpu.VMEM((K, HALF), f32),
            pltpu.VMEM((K, HALF), f32),
            pltpu.VMEM((K, HALF), f32),
            pltpu.VMEM((N,), f32),
            pltpu.SemaphoreType.DMA,
            pltpu.SemaphoreType.DMA,
            pltpu.SemaphoreType.DMA,
            pltpu.SemaphoreType.DMA,
            pltpu.VMEM_SHARED((2 * N, HALF), f32),
        ],
    )
    return kern(nf0, nf1, ed)


def _tc_body(nf, ms, avs, dxs, degp, fdig, normn, w, b, gamma, beta, out):
    deg = jnp.sum(degp[...], axis=1, keepdims=True) * 0.5
    deg = jnp.maximum(deg, 1.0)
    agg_mean = ms[...] / deg
    agg_dx = dxs[...] - fdig[...] * nf[...]
    h = (jnp.dot(nf[...], w[0:D], preferred_element_type=jnp.float32)
         + jnp.dot(agg_mean, w[D:2 * D], preferred_element_type=jnp.float32)
         + jnp.dot(avs[...], w[2 * D:3 * D], preferred_element_type=jnp.float32)
         + jnp.dot(agg_dx, w[3 * D:4 * D], preferred_element_type=jnp.float32)
         + b[...])
    h = h * normn[...]
    mu = jnp.mean(h, axis=0, keepdims=True)
    var = jnp.mean((h - mu) * (h - mu), axis=0, keepdims=True)
    h = (h - mu) * lax.rsqrt(var + 1e-5) * gamma[...] + beta[...]
    out[...] = jnp.maximum(h, 0.0)


@jax.jit
def kernel(node_fts, edge_fts, edge_index, F_norm_edge, F_dig, node_deg_vec,
           norm_n, W, b, gamma, beta):
    del edge_fts, node_deg_vec
    src = edge_index[0]
    dst = edge_index[1]
    fbits = lax.bitcast_convert_type(F_norm_edge[:, 0], jnp.int32)
    ed = jnp.stack([src, dst, dst + N, fbits])
    nf0 = node_fts[:, :HALF]
    nf1 = node_fts[:, HALF:]

    ms, avs, dxs, degp = _sc_aggregate(nf0, nf1, ed)

    out = pl.pallas_call(
        _tc_body,
        out_shape=jax.ShapeDtypeStruct((N, D), jnp.float32),
    )(node_fts, ms, avs, dxs, degp.T, F_dig, norm_n,
      W, b.reshape(1, D), gamma.reshape(1, D), beta.reshape(1, D))
    return out
